# Initial kernel scaffold; baseline (speedup 1.0000x reference)
import functools

import jax
import jax.numpy as jnp
from jax import lax
from jax.experimental import pallas as pl
from jax.experimental.pallas import tpu as pltpu
from jax.experimental.pallas import tpu_sc as plsc

VOCAB = 1000000
EMBED_DIM = 16

B_ROWS = 16384
SEQ = 200
B = B_ROWS * SEQ

NC = 2
NS = 16
NW = NC * NS
B_PER_W = B // NW
CHUNK = 2048
N_CHUNKS = B_PER_W // CHUNK


def _gather_body(table_hbm, idx_hbm, out_hbm, idx_v, rows_v, sem):
    wid = lax.axis_index("s") * NC + lax.axis_index("c")
    base = wid * B_PER_W

    def step(j, carry):
        off = base + j * CHUNK
        pltpu.sync_copy(idx_hbm.at[pl.ds(off, CHUNK)], idx_v)
        pltpu.async_copy(table_hbm.at[idx_v], rows_v, sem).wait()
        pltpu.sync_copy(rows_v, out_hbm.at[pl.ds(off, CHUNK)])
        return carry

    lax.fori_loop(0, N_CHUNKS, step, 0)


def _gather(table, idx_flat):
    mesh = plsc.VectorSubcoreMesh(core_axis_name="c", subcore_axis_name="s")
    kern = pl.kernel(
        _gather_body,
        mesh=mesh,
        out_type=jax.ShapeDtypeStruct((B, EMBED_DIM), jnp.float32),
        scratch_types=[
            pltpu.VMEM((CHUNK,), jnp.int32),
            pltpu.VMEM((CHUNK, EMBED_DIM), jnp.float32),
            pltpu.SemaphoreType.DMA,
        ],
    )
    return kern(table, idx_flat)


def kernel(table, indices):
    idx_flat = indices.reshape(B).astype(jnp.int32)
    out = _gather(table, idx_flat)
    return out.reshape(B_ROWS, SEQ, EMBED_DIM)


# speedup vs baseline: 2.4894x; 2.4894x over previous
import functools

import jax
import jax.numpy as jnp
from jax import lax
from jax.experimental import pallas as pl
from jax.experimental.pallas import tpu as pltpu
from jax.experimental.pallas import tpu_sc as plsc

VOCAB = 1000000
EMBED_DIM = 16

B_ROWS = 16384
SEQ = 200
B = B_ROWS * SEQ

NC = 2
NS = 16
NW = NC * NS
B_PER_W = B // NW
CHUNK = 2048
N_CHUNKS = B_PER_W // CHUNK


def _gather_body(table_hbm, idx_hbm, out_hbm, idx_v, rows_v, sem):
    wid = lax.axis_index("s") * NC + lax.axis_index("c")
    base = wid * B_PER_W

    def step(j, carry):
        off = base + j * CHUNK
        pltpu.sync_copy(idx_hbm.at[pl.ds(off, CHUNK)], idx_v)
        pltpu.async_copy(table_hbm.at[idx_v], rows_v, sem).wait()
        pltpu.sync_copy(rows_v, out_hbm.at[pl.ds(off, CHUNK)])
        return carry

    lax.fori_loop(0, N_CHUNKS, step, 0)


def _gather(table, idx_flat):
    mesh = plsc.VectorSubcoreMesh(core_axis_name="c", subcore_axis_name="s")
    kern = pl.kernel(
        _gather_body,
        mesh=mesh,
        out_type=jax.ShapeDtypeStruct((B, EMBED_DIM), jnp.float32),
        scratch_types=[
            pltpu.VMEM((CHUNK,), jnp.int32),
            pltpu.VMEM((CHUNK, EMBED_DIM), jnp.float32),
            pltpu.SemaphoreType.DMA,
        ],
        compiler_params=pltpu.CompilerParams(use_tc_tiling_on_sc=False),
    )
    return kern(table, idx_flat)


def kernel(table, indices):
    idx_flat = indices.reshape(B).astype(jnp.int32)
    out = _gather(table, idx_flat)
    return out.reshape(B_ROWS, SEQ, EMBED_DIM)


# speedup vs baseline: 2.5322x; 1.0172x over previous
import functools

import jax
import jax.numpy as jnp
from jax import lax
from jax.experimental import pallas as pl
from jax.experimental.pallas import tpu as pltpu
from jax.experimental.pallas import tpu_sc as plsc

VOCAB = 1000000
EMBED_DIM = 16

B_ROWS = 16384
SEQ = 200
B = B_ROWS * SEQ

NC = 2
NS = 16
NW = NC * NS
B_PER_W = B // NW
CHUNK = 2048
N_CHUNKS = B_PER_W // CHUNK


def _gather_body(table_hbm, idx_hbm, out_hbm, idx_v, rows_v, sem_g, sem_o0, sem_o1):
    wid = lax.axis_index("s") * NC + lax.axis_index("c")
    base = wid * B_PER_W
    sem_o = (sem_o0, sem_o1)

    def do_chunk(g, b, wait_out):
        off = base + g * CHUNK
        pltpu.sync_copy(idx_hbm.at[pl.ds(off, CHUNK)], idx_v.at[b])
        if wait_out:
            pltpu.make_async_copy(
                rows_v.at[b], out_hbm.at[pl.ds(base, CHUNK)], sem_o[b]
            ).wait()
        pltpu.async_copy(table_hbm.at[idx_v.at[b]], rows_v.at[b], sem_g).wait()
        pltpu.async_copy(rows_v.at[b], out_hbm.at[pl.ds(off, CHUNK)], sem_o[b])

    do_chunk(0, 0, False)
    do_chunk(1, 1, False)

    def pair(k, carry):
        do_chunk(2 * k, 0, True)
        do_chunk(2 * k + 1, 1, True)
        return carry

    lax.fori_loop(1, N_CHUNKS // 2, pair, 0)
    pltpu.make_async_copy(rows_v.at[0], out_hbm.at[pl.ds(base, CHUNK)], sem_o[0]).wait()
    pltpu.make_async_copy(rows_v.at[1], out_hbm.at[pl.ds(base, CHUNK)], sem_o[1]).wait()


def _gather(table, idx_flat):
    mesh = plsc.VectorSubcoreMesh(core_axis_name="c", subcore_axis_name="s")
    kern = pl.kernel(
        _gather_body,
        mesh=mesh,
        out_type=jax.ShapeDtypeStruct((B, EMBED_DIM), jnp.float32),
        scratch_types=[
            pltpu.VMEM((2, CHUNK), jnp.int32),
            pltpu.VMEM((2, CHUNK, EMBED_DIM), jnp.float32),
            pltpu.SemaphoreType.DMA,
            pltpu.SemaphoreType.DMA,
            pltpu.SemaphoreType.DMA,
        ],
        compiler_params=pltpu.CompilerParams(use_tc_tiling_on_sc=False),
    )
    return kern(table, idx_flat)


def kernel(table, indices):
    idx_flat = indices.reshape(B).astype(jnp.int32)
    out = _gather(table, idx_flat)
    return out.reshape(B_ROWS, SEQ, EMBED_DIM)


# speedup vs baseline: 2.5683x; 1.0143x over previous
import functools

import jax
import jax.numpy as jnp
from jax import lax
from jax.experimental import pallas as pl
from jax.experimental.pallas import tpu as pltpu
from jax.experimental.pallas import tpu_sc as plsc

VOCAB = 1000000
EMBED_DIM = 16

B_ROWS = 16384
SEQ = 200
B = B_ROWS * SEQ

NC = 2
NS = 16
NW = NC * NS
B_PER_W = B // NW
CHUNK = 2048
N_CHUNKS = B_PER_W // CHUNK


SPLIT = 4
SUB = CHUNK // SPLIT


def _gather_body(table_hbm, idx_hbm, out_hbm, idx_v, rows_v,
                 sem_g0, sem_g1, sem_o0, sem_o1):
    wid = lax.axis_index("s") * NC + lax.axis_index("c")
    base = wid * B_PER_W
    sem_g = (sem_g0, sem_g1)
    sem_o = (sem_o0, sem_o1)

    def fire_gathers(g, b):
        off = base + g * CHUNK
        pltpu.sync_copy(idx_hbm.at[pl.ds(off, CHUNK)], idx_v.at[b])
        for s in range(SPLIT):
            pltpu.async_copy(
                table_hbm.at[idx_v.at[b, pl.ds(s * SUB, SUB)]],
                rows_v.at[b, pl.ds(s * SUB, SUB)],
                sem_g[b],
            )

    def drain_and_flush(g, b, wait_out):
        off = base + g * CHUNK
        for s in range(SPLIT):
            pltpu.make_async_copy(
                table_hbm.at[idx_v.at[b, pl.ds(s * SUB, SUB)]],
                rows_v.at[b, pl.ds(s * SUB, SUB)],
                sem_g[b],
            ).wait()
        pltpu.async_copy(rows_v.at[b], out_hbm.at[pl.ds(off, CHUNK)], sem_o[b])
        if wait_out:
            pltpu.make_async_copy(
                rows_v.at[b], out_hbm.at[pl.ds(off, CHUNK)], sem_o[b]
            ).wait()

    fire_gathers(0, 0)
    fire_gathers(1, 1)

    def pair(k, carry):
        drain_and_flush(2 * k - 2, 0, True)
        fire_gathers(2 * k, 0)
        drain_and_flush(2 * k - 1, 1, True)
        fire_gathers(2 * k + 1, 1)
        return carry

    lax.fori_loop(1, N_CHUNKS // 2, pair, 0)
    drain_and_flush(N_CHUNKS - 2, 0, True)
    drain_and_flush(N_CHUNKS - 1, 1, True)


def _gather(table, idx_flat):
    mesh = plsc.VectorSubcoreMesh(core_axis_name="c", subcore_axis_name="s")
    kern = pl.kernel(
        _gather_body,
        mesh=mesh,
        out_type=jax.ShapeDtypeStruct((B, EMBED_DIM), jnp.float32),
        scratch_types=[
            pltpu.VMEM((2, CHUNK), jnp.int32),
            pltpu.VMEM((2, CHUNK, EMBED_DIM), jnp.float32),
            pltpu.SemaphoreType.DMA,
            pltpu.SemaphoreType.DMA,
            pltpu.SemaphoreType.DMA,
            pltpu.SemaphoreType.DMA,
        ],
        compiler_params=pltpu.CompilerParams(use_tc_tiling_on_sc=False),
    )
    return kern(table, idx_flat)


def kernel(table, indices):
    idx_flat = indices.reshape(B).astype(jnp.int32)
    out = _gather(table, idx_flat)
    return out.reshape(B_ROWS, SEQ, EMBED_DIM)


# speedup vs baseline: 4.9944x; 1.9446x over previous
import jax
import jax.numpy as jnp
from jax import lax
from jax.experimental import pallas as pl
from jax.experimental.pallas import tpu as pltpu
from jax.experimental.pallas import tpu_sc as plsc

VOCAB = 1000000
EMBED_DIM = 16

B_ROWS = 16384
SEQ = 200

NC = 2
NS = 16
NW = NC * NS

ST, SI = 25, 8
BT, BI = 128, 128
ET, EI = 2, 8

N_UNITS = ST * BT
U_PER_TEC = N_UNITS // NW


def _gather_body(table_hbm, idx5_hbm, out6_hbm, idxv, rows_v, t_v,
                 sem_i0, sem_i1, sem_g0, sem_g1, sem_d0, sem_d1):
    wid = lax.axis_index("s") * NC + lax.axis_index("c")
    u0 = wid * U_PER_TEC
    sem_i = (sem_i0, sem_i1)
    sem_g = (sem_g0, sem_g1)
    sem_d = (sem_d0, sem_d1)

    iota16 = lax.iota(jnp.int32, 16)
    cols = [jnp.full((16,), e, jnp.int32) for e in range(EMBED_DIM)]

    def fire_idx(k, b):
        u = u0 + k
        pltpu.async_copy(idx5_hbm.at[u // BT, u % BT], idxv.at[b], sem_i[b])

    def wait_idx(b):
        pltpu.make_async_copy(idx5_hbm.at[0, 0], idxv.at[b], sem_i[b]).wait()

    def fire_gathers(b):
        for si in range(SI):
            pltpu.async_copy(
                table_hbm.at[idxv.at[b, si]], rows_v.at[b, si], sem_g[b]
            )

    def wait_gathers(b):
        for si in range(SI):
            pltpu.make_async_copy(
                table_hbm.at[idxv.at[b, si]], rows_v.at[b, si], sem_g[b]
            ).wait()

    def transpose(b):
        def gblock(g, carry):
            rowsidx = iota16 + g * 16
            for si in range(SI):
                ref = rows_v.at[b, si]
                for e in range(EMBED_DIM):
                    vals = plsc.load_gather(ref, [rowsidx, cols[e]])
                    t_v[b, si, e // EI, e % EI, pl.ds(g * 16, 16)] = vals
            return carry

        lax.fori_loop(0, BI // 16, gblock, 0)

    def fire_outs(k, b):
        u = u0 + k
        st, bt = u // BT, u % BT
        for si in range(SI):
            for et in range(ET):
                pltpu.async_copy(
                    t_v.at[b, si, et], out6_hbm.at[st, si, et, bt], sem_d[b]
                )

    def wait_outs(b):
        for si in range(SI):
            for et in range(ET):
                pltpu.make_async_copy(
                    t_v.at[b, si, et], out6_hbm.at[0, si, et, 0], sem_d[b]
                ).wait()

    def step(k, b, nb, has_k2, has_k1, has_d):
        wait_gathers(b)
        if has_k2:
            fire_idx(k + 2, b)
        if has_k1:
            wait_idx(nb)
            fire_gathers(nb)
        if has_d:
            wait_outs(b)
        transpose(b)
        fire_outs(k, b)

    fire_idx(0, 0)
    fire_idx(1, 1)
    wait_idx(0)
    fire_gathers(0)
    step(0, 0, 1, True, True, False)
    step(1, 1, 0, True, True, False)

    def pair(j, carry):
        k = 2 * j
        step(k, 0, 1, True, True, True)
        step(k + 1, 1, 0, True, True, True)
        return carry

    lax.fori_loop(1, (U_PER_TEC - 2) // 2, pair, 0)

    step(U_PER_TEC - 2, 0, 1, False, True, True)
    step(U_PER_TEC - 1, 1, 0, False, False, True)
    wait_outs(0)
    wait_outs(1)


def _gather(table, idx5):
    mesh = plsc.VectorSubcoreMesh(core_axis_name="c", subcore_axis_name="s")
    kern = pl.kernel(
        _gather_body,
        mesh=mesh,
        out_type=jax.ShapeDtypeStruct((ST, SI, ET, BT, EI, BI), jnp.float32),
        scratch_types=[
            pltpu.VMEM((2, SI, BI), jnp.int32),
            pltpu.VMEM((2, SI, BI, EMBED_DIM), jnp.float32),
            pltpu.VMEM((2, SI, ET, EI, BI), jnp.float32),
            pltpu.SemaphoreType.DMA,
            pltpu.SemaphoreType.DMA,
            pltpu.SemaphoreType.DMA,
            pltpu.SemaphoreType.DMA,
            pltpu.SemaphoreType.DMA,
            pltpu.SemaphoreType.DMA,
        ],
        compiler_params=pltpu.CompilerParams(
            use_tc_tiling_on_sc=False, needs_layout_passes=False
        ),
    )
    return kern(table, idx5)


def kernel(table, indices):
    idx = indices.astype(jnp.int32)
    idx5 = idx.reshape(BT, BI, ST, SI).transpose(2, 0, 3, 1)
    out6 = _gather(table, idx5)
    return out6.transpose(3, 5, 0, 1, 2, 4).reshape(B_ROWS, SEQ, EMBED_DIM)


# speedup vs baseline: 8.3932x; 1.6805x over previous
import jax
import jax.numpy as jnp
from jax import lax
from jax.experimental import pallas as pl
from jax.experimental.pallas import tpu as pltpu
from jax.experimental.pallas import tpu_sc as plsc

VOCAB = 1000000
EMBED_DIM = 16

B_ROWS = 16384
SEQ = 200

NC = 2
NS = 16
NW = NC * NS

ST, SI = 25, 8
BT, BI = 128, 128
ET, EI = 2, 8

N_UNITS = ST * BT
U_PER_TEC = N_UNITS // NW


def _gather_body(table_hbm, idx5_hbm, out6_hbm, idxv, rows_v, t_v,
                 sem_i0, sem_i1, sem_g0, sem_g1, sem_d0, sem_d1):
    wid = lax.axis_index("s") * NC + lax.axis_index("c")
    u0 = wid * U_PER_TEC
    sem_i = (sem_i0, sem_i1)
    sem_g = (sem_g0, sem_g1)
    sem_d = (sem_d0, sem_d1)

    iota16 = lax.iota(jnp.int32, 16)
    cols = [jnp.full((16,), e, jnp.int32) for e in range(EMBED_DIM)]

    def fire_idx(k, b):
        u = u0 + k
        pltpu.async_copy(idx5_hbm.at[u // BT, u % BT], idxv.at[b], sem_i[b])

    def wait_idx(b):
        pltpu.make_async_copy(idx5_hbm.at[0, 0], idxv.at[b], sem_i[b]).wait()

    def fire_gathers(b):
        for si in range(SI):
            pltpu.async_copy(
                table_hbm.at[idxv.at[b, si]], rows_v.at[b, si], sem_g[b]
            )

    def wait_gathers(b):
        for si in range(SI):
            pltpu.make_async_copy(
                table_hbm.at[idxv.at[b, si]], rows_v.at[b, si], sem_g[b]
            ).wait()

    def transpose(b):
        @plsc.parallel_loop(0, SI * (BI // 16), unroll=2)
        def gblock(i):
            si = i // (BI // 16)
            g = i % (BI // 16)
            rowsidx = iota16 + g * 16
            ref = rows_v.at[b, si]
            for e in range(EMBED_DIM):
                vals = plsc.load_gather(ref, [rowsidx, cols[e]])
                t_v[b, si, e // EI, e % EI, pl.ds(g * 16, 16)] = vals

    def fire_outs(k, b):
        u = u0 + k
        st, bt = u // BT, u % BT
        for si in range(SI):
            for et in range(ET):
                pltpu.async_copy(
                    t_v.at[b, si, et], out6_hbm.at[st, si, et, bt], sem_d[b]
                )

    def wait_outs(b):
        for si in range(SI):
            for et in range(ET):
                pltpu.make_async_copy(
                    t_v.at[b, si, et], out6_hbm.at[0, si, et, 0], sem_d[b]
                ).wait()

    def step(k, b, nb, has_k2, has_k1, has_d):
        wait_gathers(b)
        if has_k2:
            fire_idx(k + 2, b)
        if has_k1:
            wait_idx(nb)
            fire_gathers(nb)
        if has_d:
            wait_outs(b)
        transpose(b)
        fire_outs(k, b)

    fire_idx(0, 0)
    fire_idx(1, 1)
    wait_idx(0)
    fire_gathers(0)
    step(0, 0, 1, True, True, False)
    step(1, 1, 0, True, True, False)

    def pair(j, carry):
        k = 2 * j
        step(k, 0, 1, True, True, True)
        step(k + 1, 1, 0, True, True, True)
        return carry

    lax.fori_loop(1, (U_PER_TEC - 2) // 2, pair, 0)

    step(U_PER_TEC - 2, 0, 1, False, True, True)
    step(U_PER_TEC - 1, 1, 0, False, False, True)
    wait_outs(0)
    wait_outs(1)


def _gather(table, idx5):
    mesh = plsc.VectorSubcoreMesh(core_axis_name="c", subcore_axis_name="s")
    kern = pl.kernel(
        _gather_body,
        mesh=mesh,
        out_type=jax.ShapeDtypeStruct((ST, SI, ET, BT, EI, BI), jnp.float32),
        scratch_types=[
            pltpu.VMEM((2, SI, BI), jnp.int32),
            pltpu.VMEM((2, SI, BI, EMBED_DIM), jnp.float32),
            pltpu.VMEM((2, SI, ET, EI, BI), jnp.float32),
            pltpu.SemaphoreType.DMA,
            pltpu.SemaphoreType.DMA,
            pltpu.SemaphoreType.DMA,
            pltpu.SemaphoreType.DMA,
            pltpu.SemaphoreType.DMA,
            pltpu.SemaphoreType.DMA,
        ],
        compiler_params=pltpu.CompilerParams(
            use_tc_tiling_on_sc=False, needs_layout_passes=False
        ),
    )
    return kern(table, idx5)


def kernel(table, indices):
    idx = indices.astype(jnp.int32)
    idx5 = idx.reshape(BT, BI, ST, SI).transpose(2, 0, 3, 1)
    out6 = _gather(table, idx5)
    return out6.transpose(3, 5, 0, 1, 2, 4).reshape(B_ROWS, SEQ, EMBED_DIM)


# speedup vs baseline: 12.0852x; 1.4399x over previous
import jax
import jax.numpy as jnp
from jax import lax
from jax.experimental import pallas as pl
from jax.experimental.pallas import tpu as pltpu
from jax.experimental.pallas import tpu_sc as plsc

VOCAB = 1000000
EMBED_DIM = 16

B_ROWS = 16384
SEQ = 200

NC = 2
NS = 16
NW = NC * NS

ST, SI = 25, 8
BT, BI = 128, 128
ET, EI = 2, 8

N_UNITS = ST * BT
U_PER_TEC = N_UNITS // NW


def _gather_body(table_hbm, idx5_hbm, out6_hbm, idxv, rows_v, t_v,
                 sem_i0, sem_i1, sem_g0, sem_g1, sem_d0, sem_d1):
    wid = lax.axis_index("s") * NC + lax.axis_index("c")
    u0 = wid * U_PER_TEC
    sem_i = (sem_i0, sem_i1)
    sem_g = (sem_g0, sem_g1)
    sem_d = (sem_d0, sem_d1)

    iota16 = lax.iota(jnp.int32, 16)
    cols = [jnp.full((16,), e, jnp.int32) for e in range(EMBED_DIM)]

    def fire_idx(k, b):
        u = u0 + k
        pltpu.async_copy(idx5_hbm.at[u // BT, u % BT], idxv.at[b], sem_i[b])

    def wait_idx(b):
        pltpu.make_async_copy(idx5_hbm.at[0, 0], idxv.at[b], sem_i[b]).wait()

    def fire_gathers(b):
        for si in range(SI):
            pltpu.async_copy(
                table_hbm.at[idxv.at[b, si]], rows_v.at[b, si], sem_g[b]
            )

    def wait_gathers(b):
        for si in range(SI):
            pltpu.make_async_copy(
                table_hbm.at[idxv.at[b, si]], rows_v.at[b, si], sem_g[b]
            ).wait()

    def transpose(b):
        @plsc.parallel_loop(0, SI * (BI // 16), unroll=2)
        def gblock(i):
            si = i // (BI // 16)
            g = i % (BI // 16)
            rowsidx = iota16 + g * 16
            ref = rows_v.at[b, si]
            for e in range(EMBED_DIM):
                vals = plsc.load_gather(ref, [rowsidx, cols[e]])
                t_v[b, si, e // EI, e % EI, pl.ds(g * 16, 16)] = vals

    def fire_outs(k, b):
        u = u0 + k
        st, bt = u // BT, u % BT
        for si in range(SI):
            for et in range(ET):
                pltpu.async_copy(
                    t_v.at[b, si, et], out6_hbm.at[st, si, et, bt], sem_d[b]
                )

    def wait_outs(b):
        for si in range(SI):
            for et in range(ET):
                pltpu.make_async_copy(
                    t_v.at[b, si, et], out6_hbm.at[0, si, et, 0], sem_d[b]
                ).wait()

    def step(k, b, nb, has_k2, has_k1, has_d):
        wait_gathers(b)
        if has_k2:
            fire_idx(k + 2, b)
        if has_k1:
            wait_idx(nb)
            fire_gathers(nb)
        if has_d:
            wait_outs(b)
        transpose(b)
        fire_outs(k, b)

    fire_idx(0, 0)
    fire_idx(1, 1)
    wait_idx(0)
    fire_gathers(0)
    step(0, 0, 1, True, True, False)
    step(1, 1, 0, True, True, False)

    def pair(j, carry):
        k = 2 * j
        step(k, 0, 1, True, True, True)
        step(k + 1, 1, 0, True, True, True)
        return carry

    lax.fori_loop(1, (U_PER_TEC - 2) // 2, pair, 0)

    step(U_PER_TEC - 2, 0, 1, False, True, True)
    step(U_PER_TEC - 1, 1, 0, False, False, True)
    wait_outs(0)
    wait_outs(1)


FULL_TILES = VOCAB // 128
TAIL = VOCAB - FULL_TILES * 128
TILES_PER_TEC = FULL_TILES // NW
EXTRA_TILES = FULL_TILES - TILES_PER_TEC * NW


def _detile_body(tabt_hbm, tail_hbm, lin_hbm, blk0, blk1, rows0, rows1,
                 sem_i0, sem_i1, sem_o0, sem_o1):
    wid = lax.axis_index("s") * NC + lax.axis_index("c")
    t0 = wid * TILES_PER_TEC
    blk = (blk0, blk1)
    rows = (rows0, rows1)
    sem_i = (sem_i0, sem_i1)
    sem_o = (sem_o0, sem_o1)

    iota16 = lax.iota(jnp.int32, 16)

    def fire_in(vt, b):
        pltpu.async_copy(
            tabt_hbm.at[:, pl.ds(vt * 128, 128)], blk[b], sem_i[b]
        )

    def wait_in(b):
        pltpu.make_async_copy(
            tabt_hbm.at[:, pl.ds(0, 128)], blk[b], sem_i[b]
        ).wait()

    def transpose(b):
        @plsc.parallel_loop(0, 128, unroll=4)
        def tbody(v):
            vals = plsc.load_gather(blk[b], [iota16, jnp.zeros((16,), jnp.int32) + v])
            rows[b][pl.ds(v * 16, 16)] = vals

    def fire_out(vt, b):
        pltpu.async_copy(
            rows[b], lin_hbm.at[pl.ds(vt * 2048, 2048)], sem_o[b]
        )

    def wait_out(b):
        pltpu.make_async_copy(
            rows[b], lin_hbm.at[pl.ds(0, 2048)], sem_o[b]
        ).wait()

    def tile_step(vt, b, nb, has_next, has_prev_out):
        wait_in(b)
        if has_next:
            fire_in(vt + 1, nb)
        if has_prev_out:
            wait_out(b)
        transpose(b)
        fire_out(vt, b)

    fire_in(t0, 0)
    tile_step(t0, 0, 1, True, False)
    tile_step(t0 + 1, 1, 0, True, False)

    def pair(j, carry):
        vt = t0 + 2 * j
        tile_step(vt, 0, 1, True, True)
        tile_step(vt + 1, 1, 0, True, True)
        return carry

    lax.fori_loop(1, TILES_PER_TEC // 2 - 1, pair, 0)
    vt_last = t0 + TILES_PER_TEC - 2
    tile_step(vt_last, 0, 1, True, True)
    tile_step(vt_last + 1, 1, 0, False, True)
    wait_out(0)
    wait_out(1)

    for w in range(EXTRA_TILES):
        @pl.when(wid == w)
        def _():
            vt = NW * TILES_PER_TEC + w
            fire_in(vt, 0)
            wait_in(0)
            transpose(0)
            fire_out(vt, 0)
            wait_out(0)

    @pl.when(wid == EXTRA_TILES)
    def _():
        pltpu.sync_copy(tail_hbm, blk0)

        @plsc.parallel_loop(0, TAIL, unroll=4)
        def tbody(v):
            vals = plsc.load_gather(blk0, [iota16, jnp.zeros((16,), jnp.int32) + v])
            rows0[pl.ds(v * 16, 16)] = vals

        pltpu.async_copy(
            rows0.at[pl.ds(0, TAIL * 16)],
            lin_hbm.at[pl.ds(FULL_TILES * 2048, TAIL * 16)],
            sem_o0,
        )
        pltpu.make_async_copy(
            rows0.at[pl.ds(0, TAIL * 16)],
            lin_hbm.at[pl.ds(0, TAIL * 16)],
            sem_o0,
        ).wait()


def _detile(table_t, tail_pad):
    mesh = plsc.VectorSubcoreMesh(core_axis_name="c", subcore_axis_name="s")
    kern = pl.kernel(
        _detile_body,
        mesh=mesh,
        out_type=jax.ShapeDtypeStruct((VOCAB * EMBED_DIM,), jnp.float32),
        scratch_types=[
            pltpu.VMEM((16, 128), jnp.float32),
            pltpu.VMEM((16, 128), jnp.float32),
            pltpu.VMEM((2048,), jnp.float32),
            pltpu.VMEM((2048,), jnp.float32),
            pltpu.SemaphoreType.DMA,
            pltpu.SemaphoreType.DMA,
            pltpu.SemaphoreType.DMA,
            pltpu.SemaphoreType.DMA,
        ],
        compiler_params=pltpu.CompilerParams(
            use_tc_tiling_on_sc=True, needs_layout_passes=False
        ),
    )
    return kern(table_t, tail_pad)


def _gather(table, idx5):
    mesh = plsc.VectorSubcoreMesh(core_axis_name="c", subcore_axis_name="s")
    kern = pl.kernel(
        _gather_body,
        mesh=mesh,
        out_type=jax.ShapeDtypeStruct((ST, SI, ET, BT, EI, BI), jnp.float32),
        scratch_types=[
            pltpu.VMEM((2, SI, BI), jnp.int32),
            pltpu.VMEM((2, SI, BI, EMBED_DIM), jnp.float32),
            pltpu.VMEM((2, SI, ET, EI, BI), jnp.float32),
            pltpu.SemaphoreType.DMA,
            pltpu.SemaphoreType.DMA,
            pltpu.SemaphoreType.DMA,
            pltpu.SemaphoreType.DMA,
            pltpu.SemaphoreType.DMA,
            pltpu.SemaphoreType.DMA,
        ],
        compiler_params=pltpu.CompilerParams(
            use_tc_tiling_on_sc=False, needs_layout_passes=False
        ),
    )
    return kern(table, idx5)


def kernel(table, indices):
    idx = indices.astype(jnp.int32)
    idx5 = idx.reshape(BT, BI, ST, SI).transpose(2, 0, 3, 1)
    tail_pad = jnp.pad(table[FULL_TILES * 128:].T, ((0, 0), (0, 128 - TAIL)))
    lin = _detile(table.T, tail_pad)
    out6 = _gather(lin.reshape(VOCAB, EMBED_DIM), idx5)
    return out6.transpose(3, 5, 0, 1, 2, 4).reshape(B_ROWS, SEQ, EMBED_DIM)
